# baseline (speedup 1.0000x reference)
import functools

import jax
import jax.numpy as jnp
from jax import lax
from jax.experimental import pallas as pl
from jax.experimental.pallas import tpu as pltpu
from jax.experimental.pallas import tpu_sc as plsc

N = 10000
NPAD = 10240
K = 16
DIN = 128
DOUT = 128
NW = 32
PPW = NPAD // NW
CP = 8
NCHUNK = PPW // CP
STRIPE = 2 * NCHUNK
A_SLOW = 48
PW = 48
AW = 4 * DIN
NB = 1024


def _splat(v, k):
    kk = jnp.full((16, 1), k, jnp.int32)
    dn = lax.GatherDimensionNumbers(
        offset_dims=(), collapsed_slice_dims=(0,), start_index_map=(0,))
    return lax.gather(v, kk, dn, (1,),
                      mode=lax.GatherScatterMode.PROMISE_IN_BOUNDS)


def _sc_gather_accum(ftab, pos3, nbr_flat):
    mesh = plsc.VectorSubcoreMesh(core_axis_name="c", subcore_axis_name="s")

    scratch = [
        pltpu.VMEM((2, CP * K), jnp.int32),
        pltpu.VMEM((NPAD,), jnp.float32),
        pltpu.VMEM((NPAD,), jnp.float32),
        pltpu.VMEM((NPAD,), jnp.float32),
        pltpu.VMEM((2, CP * K, DIN // 2), jnp.int32),
        pltpu.VMEM((4, CP, DIN), jnp.float32),
        pltpu.SemaphoreType.DMA,
        pltpu.SemaphoreType.DMA,
    ]

    @functools.partial(
        pl.kernel,
        mesh=mesh,
        out_type=jax.ShapeDtypeStruct((NPAD // 8, 4, 8, DIN), jnp.float32),
        scratch_types=scratch,
        compiler_params=pltpu.CompilerParams(needs_layout_passes=False,
                                             use_tc_tiling_on_sc=False),
    )
    def body(ftab_hbm, pos_hbm, nbr_hbm, out_hbm,
             idx_v, px_v, py_v, pz_v, frows, stage, sem0, sem1):
        s = lax.axis_index("s")
        c = lax.axis_index("c")
        chunk0 = s * STRIPE + jnp.where(c == 0, 0, A_SLOW)
        npairs = jnp.where(c == 0, A_SLOW // 2, (STRIPE - A_SLOW) // 2)
        sems = (sem0, sem1)
        pltpu.sync_copy(pos_hbm.at[pl.ds(0, NPAD)], px_v)
        pltpu.sync_copy(pos_hbm.at[pl.ds(NPAD, NPAD)], py_v)
        pltpu.sync_copy(pos_hbm.at[pl.ds(2 * NPAD, NPAD)], pz_v)

        def fetch(ch, b):
            base_pt = (chunk0 + ch) * CP
            pltpu.sync_copy(nbr_hbm.at[pl.ds(base_pt * K, CP * K)],
                            idx_v.at[b])
            return pltpu.async_copy(ftab_hbm.at[idx_v.at[b]], frows.at[b],
                                    sems[b])

        def compute(ch, b):
            crow = chunk0 + ch

            def point_body(pt, carry2):
                j0 = pt * K
                kidx = idx_v[b, pl.ds(j0, K)]
                wx = plsc.load_gather(px_v, [kidx])
                wy = plsc.load_gather(py_v, [kidx])
                wz = plsc.load_gather(pz_v, [kidx])
                acc = [jnp.zeros((16,), jnp.float32) for _ in range(32)]
                mask_hi = jnp.full((16,), -65536, jnp.int32)
                sh16 = jnp.full((16,), 16, jnp.int32)
                for k in range(K):
                    j = j0 + k
                    f = []
                    for q in range(4):
                        u = frows[b, j, pl.ds(16 * q, 16)]
                        lo = lax.shift_left(u, sh16)
                        hi = lax.bitwise_and(u, mask_hi)
                        f.append(lax.bitcast_convert_type(lo, jnp.float32))
                        f.append(lax.bitcast_convert_type(hi, jnp.float32))
                    sx, sy, sz = _splat(wx, k), _splat(wy, k), _splat(wz, k)
                    for cc in range(8):
                        acc[cc] = acc[cc] + f[cc]
                        acc[8 + cc] = acc[8 + cc] + f[cc] * sx
                        acc[16 + cc] = acc[16 + cc] + f[cc] * sy
                        acc[24 + cc] = acc[24 + cc] + f[cc] * sz
                for wg in range(4):
                    for cc in range(8):
                        stage[wg, pt, pl.ds(cc * 16, 16)] = acc[wg * 8 + cc]
                return carry2

            lax.fori_loop(0, CP, point_body, 0)
            for ct in range(4):
                pltpu.sync_copy(stage.at[ct], out_hbm.at[crow, ct])

        fetch(0, 0)

        def pair_body(i, carry):
            ch0 = 2 * i
            ch1 = ch0 + 1
            f1 = fetch(ch1, 1)
            pltpu.make_async_copy(ftab_hbm.at[idx_v.at[0]], frows.at[0],
                                  sems[0]).wait()
            compute(ch0, 0)

            @pl.when(ch1 < 2 * npairs - 1)
            def _():
                fetch(ch1 + 1, 0)

            f1.wait()
            compute(ch1, 1)
            return carry

        lax.fori_loop(0, npairs, pair_body, 0)

    return body(ftab, pos3, nbr_flat)


def _tc_contract(a4, w14, theta, posb, fbcol):
    def body(a_ref, w1_ref, th_ref, pb_ref, fb_ref, o_ref):
        dn = (((0,), (1,)), ((), ()))
        y = fb_ref[...]
        fsum = None
        for ct in range(4):
            a_ct = a_ref[:, ct, :, :].reshape(NB, DIN)
            y = y + lax.dot_general(w1_ref[ct], a_ct, dn,
                                    preferred_element_type=jnp.float32)
            if ct == 0:
                fsum = a_ct
        for p in range(3):
            tp = lax.dot_general(th_ref[p], fsum, dn,
                                 preferred_element_type=jnp.float32)
            y = y - pb_ref[p:p + 1, :] * tp
        o_ref[...] = y

    return pl.pallas_call(
        body,
        grid=(NPAD // NB,),
        in_specs=[
            pl.BlockSpec((NB // 8, 4, 8, DIN), lambda i: (i, 0, 0, 0)),
            pl.BlockSpec((4, DIN, DOUT), lambda i: (0, 0, 0)),
            pl.BlockSpec((3, DIN, DOUT), lambda i: (0, 0, 0)),
            pl.BlockSpec((8, NB), lambda i: (0, i)),
            pl.BlockSpec((DOUT, 1), lambda i: (0, 0)),
        ],
        out_specs=pl.BlockSpec((DOUT, NB), lambda i: (0, i)),
        out_shape=jax.ShapeDtypeStruct((DOUT, N), jnp.float32),
    )(a4, w14, theta, posb, fbcol)


def kernel(features, positions, neighborhoods, position_theta, position_bias,
           feature_bias):
    f = features[0]
    pos = positions[0]
    nbr = neighborhoods[0]
    theta = position_theta[0]

    perm = jnp.asarray(
        [32 * (cc // 2) + (cc % 2) + 2 * l for cc in range(8)
         for l in range(16)], dtype=jnp.int32)

    u16 = lax.bitcast_convert_type(f.astype(jnp.bfloat16), jnp.uint16)
    packed = (u16[0::2].astype(jnp.uint32)
              | (u16[1::2].astype(jnp.uint32) << 16))
    ftab = jnp.zeros((NPAD, DIN // 2), jnp.int32).at[:N].set(
        lax.bitcast_convert_type(packed, jnp.int32).T)
    pos3 = jnp.zeros((3, NPAD), jnp.float32).at[:, :N].set(pos).reshape(-1)
    nbr_flat = jnp.zeros((NPAD, K), jnp.int32).at[:N].set(nbr.T).reshape(-1)

    a4 = _sc_gather_accum(ftab, pos3, nbr_flat)

    thp = theta[:, perm, :]
    w14 = jnp.stack([position_bias[perm], thp[0], thp[1], thp[2]], axis=0)
    posb = jnp.zeros((8, NPAD), jnp.float32).at[:3, :N].set(pos)
    y = _tc_contract(a4, w14, thp, posb, feature_bias)
    return y[None]


# speedup vs baseline: 1.7111x; 1.7111x over previous
import functools

import jax
import jax.numpy as jnp
from jax import lax
from jax.experimental import pallas as pl
from jax.experimental.pallas import tpu as pltpu
from jax.experimental.pallas import tpu_sc as plsc

N = 10000
NPAD = 10240
K = 16
DIN = 128
DOUT = 128
NW = 32
PPW = NPAD // NW
CP = 8
NCHUNK = PPW // CP
STRIPE = 2 * NCHUNK
A_SLOW = 48
PW = 48
AW = 4 * DIN
NB = 1024


def _splat(v, k):
    kk = jnp.full((16, 1), k, jnp.int32)
    dn = lax.GatherDimensionNumbers(
        offset_dims=(), collapsed_slice_dims=(0,), start_index_map=(0,))
    return lax.gather(v, kk, dn, (1,),
                      mode=lax.GatherScatterMode.PROMISE_IN_BOUNDS)


def _sc_gather_accum(ftab, pos3, nbr_flat):
    mesh = plsc.VectorSubcoreMesh(core_axis_name="c", subcore_axis_name="s")

    scratch = [
        pltpu.VMEM((2, CP * K), jnp.int32),
        pltpu.VMEM((NPAD,), jnp.float32),
        pltpu.VMEM((NPAD,), jnp.float32),
        pltpu.VMEM((NPAD,), jnp.float32),
        pltpu.VMEM((2, CP * K, DIN // 2), jnp.int32),
        pltpu.VMEM((4, CP, DIN), jnp.float32),
        pltpu.SemaphoreType.DMA,
        pltpu.SemaphoreType.DMA,
    ]

    @functools.partial(
        pl.kernel,
        mesh=mesh,
        out_type=jax.ShapeDtypeStruct((NPAD // 8, 4, 8, DIN), jnp.float32),
        scratch_types=scratch,
        compiler_params=pltpu.CompilerParams(needs_layout_passes=False,
                                             use_tc_tiling_on_sc=False),
    )
    def body(ftab_hbm, pos_hbm, nbr_hbm, out_hbm,
             idx_v, px_v, py_v, pz_v, frows, stage, sem0, sem1):
        s = lax.axis_index("s")
        c = lax.axis_index("c")
        chunk0 = s * STRIPE + jnp.where(c == 0, 0, A_SLOW)
        npairs = jnp.where(c == 0, A_SLOW // 2, (STRIPE - A_SLOW) // 2)
        sems = (sem0, sem1)
        pltpu.sync_copy(pos_hbm.at[pl.ds(0, NPAD)], px_v)
        pltpu.sync_copy(pos_hbm.at[pl.ds(NPAD, NPAD)], py_v)
        pltpu.sync_copy(pos_hbm.at[pl.ds(2 * NPAD, NPAD)], pz_v)

        def fetch(ch, b):
            base_pt = (chunk0 + ch) * CP
            pltpu.sync_copy(nbr_hbm.at[pl.ds(base_pt * K, CP * K)],
                            idx_v.at[b])
            return pltpu.async_copy(ftab_hbm.at[idx_v.at[b]], frows.at[b],
                                    sems[b])

        def compute(ch, b):
            crow = chunk0 + ch

            def point_body(pt, carry2):
                j0 = pt * K
                kidx = idx_v[b, pl.ds(j0, K)]
                wx = plsc.load_gather(px_v, [kidx])
                wy = plsc.load_gather(py_v, [kidx])
                wz = plsc.load_gather(pz_v, [kidx])
                acc = [jnp.zeros((16,), jnp.float32) for _ in range(32)]
                mask_hi = jnp.full((16,), -65536, jnp.int32)
                sh16 = jnp.full((16,), 16, jnp.int32)
                for k in range(K):
                    j = j0 + k
                    f = []
                    for q in range(4):
                        u = frows[b, j, pl.ds(16 * q, 16)]
                        lo = lax.shift_left(u, sh16)
                        hi = lax.bitwise_and(u, mask_hi)
                        f.append(lax.bitcast_convert_type(lo, jnp.float32))
                        f.append(lax.bitcast_convert_type(hi, jnp.float32))
                    sx, sy, sz = _splat(wx, k), _splat(wy, k), _splat(wz, k)
                    for cc in range(8):
                        acc[cc] = acc[cc] + f[cc]
                        acc[8 + cc] = acc[8 + cc] + f[cc] * sx
                        acc[16 + cc] = acc[16 + cc] + f[cc] * sy
                        acc[24 + cc] = acc[24 + cc] + f[cc] * sz
                for wg in range(4):
                    for cc in range(8):
                        stage[wg, pt, pl.ds(cc * 16, 16)] = acc[wg * 8 + cc]
                return carry2

            lax.fori_loop(0, CP, point_body, 0)
            for ct in range(4):
                pltpu.sync_copy(stage.at[ct], out_hbm.at[crow, ct])

        fetch(0, 0)

        def pair_body(i, carry):
            ch0 = 2 * i
            ch1 = ch0 + 1
            f1 = fetch(ch1, 1)
            pltpu.make_async_copy(ftab_hbm.at[idx_v.at[0]], frows.at[0],
                                  sems[0]).wait()
            compute(ch0, 0)

            @pl.when(ch1 < 2 * npairs - 1)
            def _():
                fetch(ch1 + 1, 0)

            f1.wait()
            compute(ch1, 1)
            return carry

        lax.fori_loop(0, npairs, pair_body, 0)

    return body(ftab, pos3, nbr_flat)


def _tc_contract(a4, w14, theta, posb, fbcol):
    def body(a_ref, w1_ref, th_ref, pb_ref, fb_ref, o_ref):
        dn = (((0,), (1,)), ((), ()))
        y = fb_ref[...]
        fsum = None
        for ct in range(4):
            a_ct = a_ref[:, ct, :, :].reshape(NB, DIN)
            y = y + lax.dot_general(w1_ref[ct], a_ct, dn,
                                    preferred_element_type=jnp.float32)
            if ct == 0:
                fsum = a_ct
        for p in range(3):
            tp = lax.dot_general(th_ref[p], fsum, dn,
                                 preferred_element_type=jnp.float32)
            y = y - pb_ref[p:p + 1, :] * tp
        o_ref[...] = y

    return pl.pallas_call(
        body,
        grid=(NPAD // NB,),
        in_specs=[
            pl.BlockSpec((NB // 8, 4, 8, DIN), lambda i: (i, 0, 0, 0)),
            pl.BlockSpec((4, DIN, DOUT), lambda i: (0, 0, 0)),
            pl.BlockSpec((3, DIN, DOUT), lambda i: (0, 0, 0)),
            pl.BlockSpec((8, NB), lambda i: (0, i)),
            pl.BlockSpec((DOUT, 1), lambda i: (0, 0)),
        ],
        out_specs=pl.BlockSpec((DOUT, NB), lambda i: (0, i)),
        out_shape=jax.ShapeDtypeStruct((DOUT, N), jnp.float32),
    )(a4, w14, theta, posb, fbcol)


def kernel(features, positions, neighborhoods, position_theta, position_bias,
           feature_bias):
    f = features[0]
    pos = positions[0]
    nbr = neighborhoods[0]
    theta = position_theta[0]

    perm = jnp.asarray(
        [32 * (cc // 2) + (cc % 2) + 2 * l for cc in range(8)
         for l in range(16)], dtype=jnp.int32)

    ftab = lax.bitcast_convert_type(
        jnp.zeros((NPAD, DIN), jnp.bfloat16).at[:N].set(
            f.T.astype(jnp.bfloat16)).reshape(NPAD, DIN // 2, 2),
        jnp.int32)
    pos3 = jnp.zeros((3, NPAD), jnp.float32).at[:, :N].set(pos).reshape(-1)
    nbr_flat = jnp.zeros((NPAD, K), jnp.int32).at[:N].set(nbr.T).reshape(-1)

    a4 = _sc_gather_accum(ftab, pos3, nbr_flat)

    thp = theta[:, perm, :]
    w14 = jnp.stack([position_bias[perm], thp[0], thp[1], thp[2]], axis=0)
    posb = jnp.zeros((8, NPAD), jnp.float32).at[:3, :N].set(pos)
    y = _tc_contract(a4, w14, thp, posb, feature_bias)
    return y[None]
